# Initial kernel scaffold; baseline (speedup 1.0000x reference)
import jax
import jax.numpy as jnp
from jax.experimental import pallas as pl


def kernel(x, adj, W_heads, a_src_heads, a_dst_heads, b_heads, W_out, a_src_out, a_dst_out, b_out):
    raise NotImplementedError("write your pallas kernel here")


# speedup vs baseline: 7459.6469x; 7459.6469x over previous
import jax
import jax.numpy as jnp
from jax.experimental import pallas as pl

N = 1024
IN_FEAT = 128
N_HIDDEN = 64
N_HEADS = 8
OUT_FEAT = 64
NEG_SLOPE = 0.2
BJ = 256


def _attention(h, hb, adj_blk, a_s, a_d, b, jb):
    s = jax.lax.dot_general(h, a_s, (((1,), (1,)), ((), ())),
                            preferred_element_type=jnp.float32)
    d = jax.lax.dot_general(a_d, hb, (((1,), (1,)), ((), ())),
                            preferred_element_type=jnp.float32)
    e = s + d
    e = jnp.where(e > 0, e, NEG_SLOPE * e)
    ii = jax.lax.broadcasted_iota(jnp.int32, (N, BJ), 0)
    jj = jax.lax.broadcasted_iota(jnp.int32, (N, BJ), 1) + jb * BJ
    valid = ((adj_blk != 0) & (ii != jj)) | (ii == jj)
    e = jnp.where(valid, e, -jnp.inf)
    m = jnp.max(e, axis=0, keepdims=True)
    p = jnp.exp(e - m)
    denom = jnp.sum(p, axis=0, keepdims=True)
    alpha = p / (denom + 1e-16)
    out = jax.lax.dot_general(alpha, h, (((0,), (0,)), ((), ())),
                              preferred_element_type=jnp.float32)
    return out + b


def _layer1_kernel(x_ref, xb_ref, adj_ref, W_ref, as_ref, ad_ref, b_ref, out_ref):
    jb = pl.program_id(0)
    W = W_ref[0]
    h = jnp.dot(x_ref[...], W, preferred_element_type=jnp.float32)
    hb = jnp.dot(xb_ref[...], W, preferred_element_type=jnp.float32)
    out_ref[0] = _attention(h, hb, adj_ref[...],
                            as_ref[0], ad_ref[0], b_ref[0], jb)


def _layer2_kernel(hc_ref, hcb_ref, adj_ref, W_ref, as_ref, ad_ref, b_ref, out_ref):
    jb = pl.program_id(0)
    h = jnp.dot(hc_ref[0], W_ref[0], preferred_element_type=jnp.float32)
    hb = jnp.dot(hcb_ref[0], W_ref[0], preferred_element_type=jnp.float32)
    for k in range(1, N_HEADS):
        h = h + jnp.dot(hc_ref[k], W_ref[k], preferred_element_type=jnp.float32)
        hb = hb + jnp.dot(hcb_ref[k], W_ref[k], preferred_element_type=jnp.float32)
    o = _attention(h, hb, adj_ref[...], as_ref[...], ad_ref[...], b_ref[...], jb)
    o = jnp.where(o > 0, o, jnp.exp(o) - 1.0)
    mm = jnp.max(o, axis=1, keepdims=True)
    z = o - mm
    lse = jnp.log(jnp.sum(jnp.exp(z), axis=1, keepdims=True))
    out_ref[...] = z - lse


def kernel(x, adj, W_heads, a_src_heads, a_dst_heads, b_heads,
           W_out, a_src_out, a_dst_out, b_out):
    adj = adj.astype(jnp.int32)
    njb = N // BJ

    hheads = pl.pallas_call(
        _layer1_kernel,
        grid=(njb, N_HEADS),
        in_specs=[
            pl.BlockSpec((N, IN_FEAT), lambda jb, h: (0, 0)),
            pl.BlockSpec((BJ, IN_FEAT), lambda jb, h: (jb, 0)),
            pl.BlockSpec((N, BJ), lambda jb, h: (0, jb)),
            pl.BlockSpec((1, IN_FEAT, N_HIDDEN), lambda jb, h: (h, 0, 0)),
            pl.BlockSpec((1, 1, N_HIDDEN), lambda jb, h: (h, 0, 0)),
            pl.BlockSpec((1, 1, N_HIDDEN), lambda jb, h: (h, 0, 0)),
            pl.BlockSpec((1, 1, N_HIDDEN), lambda jb, h: (h, 0, 0)),
        ],
        out_specs=pl.BlockSpec((1, BJ, N_HIDDEN), lambda jb, h: (h, jb, 0)),
        out_shape=jax.ShapeDtypeStruct((N_HEADS, N, N_HIDDEN), jnp.float32),
    )(x, x, adj, W_heads,
      a_src_heads.reshape(N_HEADS, 1, N_HIDDEN),
      a_dst_heads.reshape(N_HEADS, 1, N_HIDDEN),
      b_heads.reshape(N_HEADS, 1, N_HIDDEN))

    out = pl.pallas_call(
        _layer2_kernel,
        grid=(njb,),
        in_specs=[
            pl.BlockSpec((N_HEADS, N, N_HIDDEN), lambda jb: (0, 0, 0)),
            pl.BlockSpec((N_HEADS, BJ, N_HIDDEN), lambda jb: (0, jb, 0)),
            pl.BlockSpec((N, BJ), lambda jb: (0, jb)),
            pl.BlockSpec((N_HEADS, N_HIDDEN, OUT_FEAT), lambda jb: (0, 0, 0)),
            pl.BlockSpec((1, OUT_FEAT), lambda jb: (0, 0)),
            pl.BlockSpec((1, OUT_FEAT), lambda jb: (0, 0)),
            pl.BlockSpec((1, OUT_FEAT), lambda jb: (0, 0)),
        ],
        out_specs=pl.BlockSpec((BJ, OUT_FEAT), lambda jb: (jb, 0)),
        out_shape=jax.ShapeDtypeStruct((N, OUT_FEAT), jnp.float32),
    )(hheads, hheads, adj,
      W_out.reshape(N_HEADS, N_HIDDEN, OUT_FEAT),
      a_src_out.reshape(1, OUT_FEAT), a_dst_out.reshape(1, OUT_FEAT),
      b_out.reshape(1, OUT_FEAT))
    return out


# speedup vs baseline: 8704.3374x; 1.1669x over previous
import jax
import jax.numpy as jnp
from jax.experimental import pallas as pl

N = 1024
IN_FEAT = 128
N_HIDDEN = 64
N_HEADS = 8
FCAT = N_HIDDEN * N_HEADS
OUT_FEAT = 64
NEG_SLOPE = 0.2
BJ = 256
NEG_INF = float("-inf")


def _prep_kernel(x_ref, Wcat_ref, Asrc_ref, Adst_ref,
                 hproj_ref, sall_ref, dallT_ref):
    hproj = jnp.dot(x_ref[...], Wcat_ref[...],
                    preferred_element_type=jnp.float32)
    hproj_ref[...] = hproj
    sall_ref[...] = jnp.dot(hproj, Asrc_ref[...],
                            preferred_element_type=jnp.float32)
    dallT_ref[...] = jax.lax.dot_general(
        Adst_ref[...], hproj, (((0,), (1,)), ((), ())),
        preferred_element_type=jnp.float32)


def _softmax_agg(e_masked, h):
    m = jnp.max(e_masked, axis=0, keepdims=True)
    p = jnp.exp(e_masked - m)
    ones = jnp.ones((N, 1), dtype=jnp.float32)
    denom = jax.lax.dot_general(p, ones, (((0,), (0,)), ((), ())),
                                preferred_element_type=jnp.float32)
    acc = jax.lax.dot_general(p, h, (((0,), (0,)), ((), ())),
                              preferred_element_type=jnp.float32)
    return acc / (denom + 1e-16)


def _layer1_kernel(hproj_ref, sall_ref, dallT_ref, adj_ref, b_ref, out_ref):
    jb = pl.program_id(0)
    ii = jax.lax.broadcasted_iota(jnp.int32, (N, BJ), 0)
    jj = jax.lax.broadcasted_iota(jnp.int32, (N, BJ), 1) + jb * BJ
    valid = ((adj_ref[...] != 0) & (ii != jj)) | (ii == jj)
    bias = jnp.where(valid, 0.0, NEG_INF)
    hproj = hproj_ref[...]
    for k in range(N_HEADS):
        s = sall_ref[:, k:k + 1]
        d = dallT_ref[k:k + 1, :]
        e = s + d
        e = jnp.where(e > 0, e, NEG_SLOPE * e) + bias
        h = hproj[:, k * N_HIDDEN:(k + 1) * N_HIDDEN]
        out = _softmax_agg(e, h) + b_ref[0, k:k + 1, :]
        out_ref[:, k * N_HIDDEN:(k + 1) * N_HIDDEN] = out


def _layer2_kernel(hc_ref, hcb_ref, adj_ref, W_ref, as_ref, ad_ref, b_ref,
                   out_ref):
    jb = pl.program_id(0)
    h = jnp.dot(hc_ref[...], W_ref[...],
                preferred_element_type=jnp.float32)
    hb = jnp.dot(hcb_ref[...], W_ref[...],
                 preferred_element_type=jnp.float32)
    s = jax.lax.dot_general(h, as_ref[...], (((1,), (1,)), ((), ())),
                            preferred_element_type=jnp.float32)
    d = jax.lax.dot_general(ad_ref[...], hb, (((1,), (1,)), ((), ())),
                            preferred_element_type=jnp.float32)
    e = s + d
    e = jnp.where(e > 0, e, NEG_SLOPE * e)
    ii = jax.lax.broadcasted_iota(jnp.int32, (N, BJ), 0)
    jj = jax.lax.broadcasted_iota(jnp.int32, (N, BJ), 1) + jb * BJ
    valid = ((adj_ref[...] != 0) & (ii != jj)) | (ii == jj)
    e = jnp.where(valid, e, NEG_INF)
    o = _softmax_agg(e, h) + b_ref[...]
    o = jnp.where(o > 0, o, jnp.exp(o) - 1.0)
    mm = jnp.max(o, axis=1, keepdims=True)
    z = o - mm
    lse = jnp.log(jnp.sum(jnp.exp(z), axis=1, keepdims=True))
    out_ref[...] = z - lse


def kernel(x, adj, W_heads, a_src_heads, a_dst_heads, b_heads,
           W_out, a_src_out, a_dst_out, b_out):
    adj = adj.astype(jnp.int32)
    njb = N // BJ

    Wcat = jnp.transpose(W_heads, (1, 0, 2)).reshape(IN_FEAT, FCAT)
    eye = jnp.eye(N_HEADS, dtype=jnp.float32)
    Asrc = (a_src_heads[:, :, None] * eye[:, None, :]).reshape(FCAT, N_HEADS)
    Adst = (a_dst_heads[:, :, None] * eye[:, None, :]).reshape(FCAT, N_HEADS)

    hproj, s_all, d_allT = pl.pallas_call(
        _prep_kernel,
        in_specs=[
            pl.BlockSpec((N, IN_FEAT), lambda: (0, 0)),
            pl.BlockSpec((IN_FEAT, FCAT), lambda: (0, 0)),
            pl.BlockSpec((FCAT, N_HEADS), lambda: (0, 0)),
            pl.BlockSpec((FCAT, N_HEADS), lambda: (0, 0)),
        ],
        out_specs=[
            pl.BlockSpec((N, FCAT), lambda: (0, 0)),
            pl.BlockSpec((N, N_HEADS), lambda: (0, 0)),
            pl.BlockSpec((N_HEADS, N), lambda: (0, 0)),
        ],
        out_shape=[
            jax.ShapeDtypeStruct((N, FCAT), jnp.float32),
            jax.ShapeDtypeStruct((N, N_HEADS), jnp.float32),
            jax.ShapeDtypeStruct((N_HEADS, N), jnp.float32),
        ],
    )(x, Wcat, Asrc, Adst)

    hcat = pl.pallas_call(
        _layer1_kernel,
        grid=(njb,),
        in_specs=[
            pl.BlockSpec((N, FCAT), lambda jb: (0, 0)),
            pl.BlockSpec((N, N_HEADS), lambda jb: (0, 0)),
            pl.BlockSpec((N_HEADS, BJ), lambda jb: (0, jb)),
            pl.BlockSpec((N, BJ), lambda jb: (0, jb)),
            pl.BlockSpec((1, N_HEADS, N_HIDDEN), lambda jb: (0, 0, 0)),
        ],
        out_specs=pl.BlockSpec((BJ, FCAT), lambda jb: (jb, 0)),
        out_shape=jax.ShapeDtypeStruct((N, FCAT), jnp.float32),
    )(hproj, s_all, d_allT, adj, b_heads.reshape(1, N_HEADS, N_HIDDEN))

    out = pl.pallas_call(
        _layer2_kernel,
        grid=(njb,),
        in_specs=[
            pl.BlockSpec((N, FCAT), lambda jb: (0, 0)),
            pl.BlockSpec((BJ, FCAT), lambda jb: (jb, 0)),
            pl.BlockSpec((N, BJ), lambda jb: (0, jb)),
            pl.BlockSpec((FCAT, OUT_FEAT), lambda jb: (0, 0)),
            pl.BlockSpec((1, OUT_FEAT), lambda jb: (0, 0)),
            pl.BlockSpec((1, OUT_FEAT), lambda jb: (0, 0)),
            pl.BlockSpec((1, OUT_FEAT), lambda jb: (0, 0)),
        ],
        out_specs=pl.BlockSpec((BJ, OUT_FEAT), lambda jb: (jb, 0)),
        out_shape=jax.ShapeDtypeStruct((N, OUT_FEAT), jnp.float32),
    )(hcat, hcat, adj, W_out,
      a_src_out.reshape(1, OUT_FEAT), a_dst_out.reshape(1, OUT_FEAT),
      b_out.reshape(1, OUT_FEAT))
    return out


# speedup vs baseline: 10544.6935x; 1.2114x over previous
import jax
import jax.numpy as jnp
from jax.experimental import pallas as pl

N = 1024
IN_FEAT = 128
N_HIDDEN = 64
N_HEADS = 8
FCAT = N_HIDDEN * N_HEADS
OUT_FEAT = 64
NEG_SLOPE = 0.2
BJ = 256


def _prep_kernel(x_ref, Wcat_ref, Asrc_ref, Adst_ref,
                 hproj_ref, sall_ref, dallT_ref):
    hproj = jnp.dot(x_ref[...], Wcat_ref[...],
                    preferred_element_type=jnp.float32)
    hproj_ref[...] = hproj
    sall_ref[...] = jnp.dot(hproj, Asrc_ref[...],
                            preferred_element_type=jnp.float32)
    dallT_ref[...] = jax.lax.dot_general(
        Adst_ref[...], hproj, (((0,), (1,)), ((), ())),
        preferred_element_type=jnp.float32)


def _mask01(adj_blk, jb):
    ii = jax.lax.broadcasted_iota(jnp.int32, (N, BJ), 0)
    jj = jax.lax.broadcasted_iota(jnp.int32, (N, BJ), 1) + jb * BJ
    valid = ((adj_blk != 0) & (ii != jj)) | (ii == jj)
    return jnp.where(valid, 1.0, 0.0)


def _prob(s, d, mask01):
    smax = jnp.max(s, axis=0, keepdims=True)
    t = smax + d
    mrow = jnp.where(t > 0, t, NEG_SLOPE * t)
    d1 = d - mrow
    d2 = NEG_SLOPE * d - mrow
    p = jnp.exp(jnp.maximum(s + d1, NEG_SLOPE * s + d2))
    return p * mask01


def _layer1_kernel(hproj_ref, sall_ref, dallT_ref, adj_ref, bT_ref, out_ref):
    jb = pl.program_id(0)
    mask01 = _mask01(adj_ref[...], jb)
    hproj = hproj_ref[...]
    for k in range(N_HEADS):
        s = sall_ref[:, k:k + 1]
        d = dallT_ref[k:k + 1, :]
        p = _prob(s, d, mask01)
        denom = jnp.sum(p, axis=0, keepdims=True)
        h = hproj[:, k * N_HIDDEN:(k + 1) * N_HIDDEN]
        accT = jax.lax.dot_general(h, p, (((0,), (0,)), ((), ())),
                                   preferred_element_type=jnp.float32)
        outT = accT * (1.0 / (denom + 1e-16)) + bT_ref[:, k:k + 1]
        out_ref[k * N_HIDDEN:(k + 1) * N_HIDDEN, :] = outT


def _layer2_kernel(hcT_ref, hcTb_ref, adj_ref, W_ref, as_ref, ad_ref, b_ref,
                   out_ref):
    jb = pl.program_id(0)
    h = jax.lax.dot_general(hcT_ref[...], W_ref[...], (((0,), (0,)), ((), ())),
                            preferred_element_type=jnp.float32)
    hb = jax.lax.dot_general(hcTb_ref[...], W_ref[...], (((0,), (0,)), ((), ())),
                             preferred_element_type=jnp.float32)
    s = jax.lax.dot_general(h, as_ref[...], (((1,), (1,)), ((), ())),
                            preferred_element_type=jnp.float32)
    d = jax.lax.dot_general(ad_ref[...], hb, (((1,), (1,)), ((), ())),
                            preferred_element_type=jnp.float32)
    p = _prob(s, d, _mask01(adj_ref[...], jb))
    denom = jnp.sum(p, axis=0, keepdims=True)
    acc = jax.lax.dot_general(p, h, (((0,), (0,)), ((), ())),
                              preferred_element_type=jnp.float32)
    recip_col = jnp.transpose(1.0 / (denom + 1e-16))
    o = acc * recip_col + b_ref[...]
    o = jnp.where(o > 0, o, jnp.exp(o) - 1.0)
    mm = jnp.max(o, axis=1, keepdims=True)
    z = o - mm
    lse = jnp.log(jnp.sum(jnp.exp(z), axis=1, keepdims=True))
    out_ref[...] = z - lse


def kernel(x, adj, W_heads, a_src_heads, a_dst_heads, b_heads,
           W_out, a_src_out, a_dst_out, b_out):
    adj = adj.astype(jnp.int32)
    njb = N // BJ

    Wcat = jnp.transpose(W_heads, (1, 0, 2)).reshape(IN_FEAT, FCAT)
    eye = jnp.eye(N_HEADS, dtype=jnp.float32)
    Asrc = (a_src_heads[:, :, None] * eye[:, None, :]).reshape(FCAT, N_HEADS)
    Adst = (a_dst_heads[:, :, None] * eye[:, None, :]).reshape(FCAT, N_HEADS)

    hproj, s_all, d_allT = pl.pallas_call(
        _prep_kernel,
        in_specs=[
            pl.BlockSpec((N, IN_FEAT), lambda: (0, 0)),
            pl.BlockSpec((IN_FEAT, FCAT), lambda: (0, 0)),
            pl.BlockSpec((FCAT, N_HEADS), lambda: (0, 0)),
            pl.BlockSpec((FCAT, N_HEADS), lambda: (0, 0)),
        ],
        out_specs=[
            pl.BlockSpec((N, FCAT), lambda: (0, 0)),
            pl.BlockSpec((N, N_HEADS), lambda: (0, 0)),
            pl.BlockSpec((N_HEADS, N), lambda: (0, 0)),
        ],
        out_shape=[
            jax.ShapeDtypeStruct((N, FCAT), jnp.float32),
            jax.ShapeDtypeStruct((N, N_HEADS), jnp.float32),
            jax.ShapeDtypeStruct((N_HEADS, N), jnp.float32),
        ],
    )(x, Wcat, Asrc, Adst)

    hcatT = pl.pallas_call(
        _layer1_kernel,
        grid=(njb,),
        in_specs=[
            pl.BlockSpec((N, FCAT), lambda jb: (0, 0)),
            pl.BlockSpec((N, N_HEADS), lambda jb: (0, 0)),
            pl.BlockSpec((N_HEADS, BJ), lambda jb: (0, jb)),
            pl.BlockSpec((N, BJ), lambda jb: (0, jb)),
            pl.BlockSpec((N_HIDDEN, N_HEADS), lambda jb: (0, 0)),
        ],
        out_specs=pl.BlockSpec((FCAT, BJ), lambda jb: (0, jb)),
        out_shape=jax.ShapeDtypeStruct((FCAT, N), jnp.float32),
    )(hproj, s_all, d_allT, adj, jnp.transpose(b_heads))

    out = pl.pallas_call(
        _layer2_kernel,
        grid=(njb,),
        in_specs=[
            pl.BlockSpec((FCAT, N), lambda jb: (0, 0)),
            pl.BlockSpec((FCAT, BJ), lambda jb: (0, jb)),
            pl.BlockSpec((N, BJ), lambda jb: (0, jb)),
            pl.BlockSpec((FCAT, OUT_FEAT), lambda jb: (0, 0)),
            pl.BlockSpec((1, OUT_FEAT), lambda jb: (0, 0)),
            pl.BlockSpec((1, OUT_FEAT), lambda jb: (0, 0)),
            pl.BlockSpec((1, OUT_FEAT), lambda jb: (0, 0)),
        ],
        out_specs=pl.BlockSpec((BJ, OUT_FEAT), lambda jb: (jb, 0)),
        out_shape=jax.ShapeDtypeStruct((N, OUT_FEAT), jnp.float32),
    )(hcatT, hcatT, adj, W_out,
      a_src_out.reshape(1, OUT_FEAT), a_dst_out.reshape(1, OUT_FEAT),
      b_out.reshape(1, OUT_FEAT))
    return out


# speedup vs baseline: 12054.4084x; 1.1432x over previous
import jax
import jax.numpy as jnp
from jax.experimental import pallas as pl

N = 1024
IN_FEAT = 128
N_HIDDEN = 64
N_HEADS = 8
FCAT = N_HIDDEN * N_HEADS
OUT_FEAT = 64
NEG_SLOPE = 0.2
BJ = 512


def _prep_kernel(x_ref, Wcat_ref, Asrc_ref, Adst_ref,
                 hproj_ref, sall_ref, dallT_ref):
    hproj = jnp.dot(x_ref[...], Wcat_ref[...],
                    preferred_element_type=jnp.float32)
    hproj_ref[...] = hproj
    sall_ref[...] = jnp.dot(hproj, Asrc_ref[...],
                            preferred_element_type=jnp.float32)
    dallT_ref[...] = jax.lax.dot_general(
        Adst_ref[...], hproj, (((0,), (1,)), ((), ())),
        preferred_element_type=jnp.float32)


def _mask01(adj_blk, jb):
    ii = jax.lax.broadcasted_iota(jnp.int32, (N, BJ), 0)
    jj = jax.lax.broadcasted_iota(jnp.int32, (N, BJ), 1) + jb * BJ
    valid = ((adj_blk != 0) & (ii != jj)) | (ii == jj)
    return jnp.where(valid, 1.0, 0.0)


def _prob(s, d, mask01):
    smax = jnp.max(s, axis=0, keepdims=True)
    t = smax + d
    mrow = jnp.where(t > 0, t, NEG_SLOPE * t)
    d1 = d - mrow
    d2 = NEG_SLOPE * d - mrow
    p = jnp.exp(jnp.maximum(s + d1, NEG_SLOPE * s + d2))
    return p * mask01


def _layer1_kernel(hproj_ref, sall_ref, dallT_ref, adj_ref, bT_ref, out_ref):
    jb = pl.program_id(0)
    mask01 = _mask01(adj_ref[...], jb)
    hproj = hproj_ref[...]
    for k in range(N_HEADS):
        s = sall_ref[:, k:k + 1]
        d = dallT_ref[k:k + 1, :]
        p = _prob(s, d, mask01)
        denom = jnp.sum(p, axis=0, keepdims=True)
        h = hproj[:, k * N_HIDDEN:(k + 1) * N_HIDDEN]
        accT = jax.lax.dot_general(h, p, (((0,), (0,)), ((), ())),
                                   preferred_element_type=jnp.float32)
        outT = accT * (1.0 / (denom + 1e-16)) + bT_ref[:, k:k + 1]
        out_ref[k * N_HIDDEN:(k + 1) * N_HIDDEN, :] = outT


def _layer2_kernel(hcT_ref, hcTb_ref, adj_ref, W_ref, as_ref, ad_ref, b_ref,
                   out_ref):
    jb = pl.program_id(0)
    h = jax.lax.dot_general(hcT_ref[...], W_ref[...], (((0,), (0,)), ((), ())),
                            preferred_element_type=jnp.float32)
    hb = jax.lax.dot_general(hcTb_ref[...], W_ref[...], (((0,), (0,)), ((), ())),
                             preferred_element_type=jnp.float32)
    s = jax.lax.dot_general(h, as_ref[...], (((1,), (1,)), ((), ())),
                            preferred_element_type=jnp.float32)
    d = jax.lax.dot_general(ad_ref[...], hb, (((1,), (1,)), ((), ())),
                            preferred_element_type=jnp.float32)
    p = _prob(s, d, _mask01(adj_ref[...], jb))
    denom = jnp.sum(p, axis=0, keepdims=True)
    acc = jax.lax.dot_general(p, h, (((0,), (0,)), ((), ())),
                              preferred_element_type=jnp.float32)
    recip_col = jnp.transpose(1.0 / (denom + 1e-16))
    o = acc * recip_col + b_ref[...]
    o = jnp.where(o > 0, o, jnp.exp(o) - 1.0)
    mm = jnp.max(o, axis=1, keepdims=True)
    z = o - mm
    lse = jnp.log(jnp.sum(jnp.exp(z), axis=1, keepdims=True))
    out_ref[...] = z - lse


def kernel(x, adj, W_heads, a_src_heads, a_dst_heads, b_heads,
           W_out, a_src_out, a_dst_out, b_out):
    adj = adj.astype(jnp.int32)
    njb = N // BJ

    Wcat = jnp.transpose(W_heads, (1, 0, 2)).reshape(IN_FEAT, FCAT)
    eye = jnp.eye(N_HEADS, dtype=jnp.float32)
    Asrc = (a_src_heads[:, :, None] * eye[:, None, :]).reshape(FCAT, N_HEADS)
    Adst = (a_dst_heads[:, :, None] * eye[:, None, :]).reshape(FCAT, N_HEADS)

    hproj, s_all, d_allT = pl.pallas_call(
        _prep_kernel,
        in_specs=[
            pl.BlockSpec((N, IN_FEAT), lambda: (0, 0)),
            pl.BlockSpec((IN_FEAT, FCAT), lambda: (0, 0)),
            pl.BlockSpec((FCAT, N_HEADS), lambda: (0, 0)),
            pl.BlockSpec((FCAT, N_HEADS), lambda: (0, 0)),
        ],
        out_specs=[
            pl.BlockSpec((N, FCAT), lambda: (0, 0)),
            pl.BlockSpec((N, N_HEADS), lambda: (0, 0)),
            pl.BlockSpec((N_HEADS, N), lambda: (0, 0)),
        ],
        out_shape=[
            jax.ShapeDtypeStruct((N, FCAT), jnp.float32),
            jax.ShapeDtypeStruct((N, N_HEADS), jnp.float32),
            jax.ShapeDtypeStruct((N_HEADS, N), jnp.float32),
        ],
    )(x, Wcat, Asrc, Adst)

    hcatT = pl.pallas_call(
        _layer1_kernel,
        grid=(njb,),
        in_specs=[
            pl.BlockSpec((N, FCAT), lambda jb: (0, 0)),
            pl.BlockSpec((N, N_HEADS), lambda jb: (0, 0)),
            pl.BlockSpec((N_HEADS, BJ), lambda jb: (0, jb)),
            pl.BlockSpec((N, BJ), lambda jb: (0, jb)),
            pl.BlockSpec((N_HIDDEN, N_HEADS), lambda jb: (0, 0)),
        ],
        out_specs=pl.BlockSpec((FCAT, BJ), lambda jb: (0, jb)),
        out_shape=jax.ShapeDtypeStruct((FCAT, N), jnp.float32),
    )(hproj, s_all, d_allT, adj, jnp.transpose(b_heads))

    out = pl.pallas_call(
        _layer2_kernel,
        grid=(njb,),
        in_specs=[
            pl.BlockSpec((FCAT, N), lambda jb: (0, 0)),
            pl.BlockSpec((FCAT, BJ), lambda jb: (0, jb)),
            pl.BlockSpec((N, BJ), lambda jb: (0, jb)),
            pl.BlockSpec((FCAT, OUT_FEAT), lambda jb: (0, 0)),
            pl.BlockSpec((1, OUT_FEAT), lambda jb: (0, 0)),
            pl.BlockSpec((1, OUT_FEAT), lambda jb: (0, 0)),
            pl.BlockSpec((1, OUT_FEAT), lambda jb: (0, 0)),
        ],
        out_specs=pl.BlockSpec((BJ, OUT_FEAT), lambda jb: (jb, 0)),
        out_shape=jax.ShapeDtypeStruct((N, OUT_FEAT), jnp.float32),
    )(hcatT, hcatT, adj, W_out,
      a_src_out.reshape(1, OUT_FEAT), a_dst_out.reshape(1, OUT_FEAT),
      b_out.reshape(1, OUT_FEAT))
    return out


# speedup vs baseline: 13027.1057x; 1.0807x over previous
import jax
import jax.numpy as jnp
from jax.experimental import pallas as pl

N = 1024
IN_FEAT = 128
N_HIDDEN = 64
N_HEADS = 8
FCAT = N_HIDDEN * N_HEADS
OUT_FEAT = 64
NEG_SLOPE = 0.2
BJ = 1024


def _prep_kernel(x_ref, Wcat_ref, Asrc_ref, Adst_ref,
                 hproj_ref, sall_ref, dallT_ref):
    hproj = jnp.dot(x_ref[...], Wcat_ref[...],
                    preferred_element_type=jnp.float32)
    hproj_ref[...] = hproj
    sall_ref[...] = jnp.dot(hproj, Asrc_ref[...],
                            preferred_element_type=jnp.float32)
    dallT_ref[...] = jax.lax.dot_general(
        Adst_ref[...], hproj, (((0,), (1,)), ((), ())),
        preferred_element_type=jnp.float32)


def _mask01(adj_blk, jb):
    ii = jax.lax.broadcasted_iota(jnp.int32, (N, BJ), 0)
    jj = jax.lax.broadcasted_iota(jnp.int32, (N, BJ), 1) + jb * BJ
    valid = ((adj_blk != 0) & (ii != jj)) | (ii == jj)
    return jnp.where(valid, 1.0, 0.0)


def _prob(s, d, mask01):
    smax = jnp.max(s, axis=0, keepdims=True)
    t = smax + d
    mrow = jnp.where(t > 0, t, NEG_SLOPE * t)
    d1 = d - mrow
    d2 = NEG_SLOPE * d - mrow
    p = jnp.exp(jnp.maximum(s + d1, NEG_SLOPE * s + d2))
    return p * mask01


def _layer1_kernel(hproj_ref, sall_ref, dallT_ref, adj_ref, bT_ref, out_ref):
    jb = pl.program_id(0)
    mask01 = _mask01(adj_ref[...], jb)
    hproj = hproj_ref[...]
    for k in range(N_HEADS):
        s = sall_ref[:, k:k + 1]
        d = dallT_ref[k:k + 1, :]
        p = _prob(s, d, mask01)
        denom = jnp.sum(p, axis=0, keepdims=True)
        h = hproj[:, k * N_HIDDEN:(k + 1) * N_HIDDEN]
        accT = jax.lax.dot_general(h, p, (((0,), (0,)), ((), ())),
                                   preferred_element_type=jnp.float32)
        outT = accT * (1.0 / (denom + 1e-16)) + bT_ref[:, k:k + 1]
        out_ref[k * N_HIDDEN:(k + 1) * N_HIDDEN, :] = outT


def _layer2_kernel(hcT_ref, hcTb_ref, adj_ref, W_ref, as_ref, ad_ref, b_ref,
                   out_ref):
    jb = pl.program_id(0)
    h = jax.lax.dot_general(hcT_ref[...], W_ref[...], (((0,), (0,)), ((), ())),
                            preferred_element_type=jnp.float32)
    hb = jax.lax.dot_general(hcTb_ref[...], W_ref[...], (((0,), (0,)), ((), ())),
                             preferred_element_type=jnp.float32)
    s = jax.lax.dot_general(h, as_ref[...], (((1,), (1,)), ((), ())),
                            preferred_element_type=jnp.float32)
    d = jax.lax.dot_general(ad_ref[...], hb, (((1,), (1,)), ((), ())),
                            preferred_element_type=jnp.float32)
    p = _prob(s, d, _mask01(adj_ref[...], jb))
    denom = jnp.sum(p, axis=0, keepdims=True)
    acc = jax.lax.dot_general(p, h, (((0,), (0,)), ((), ())),
                              preferred_element_type=jnp.float32)
    recip_col = jnp.transpose(1.0 / (denom + 1e-16))
    o = acc * recip_col + b_ref[...]
    o = jnp.where(o > 0, o, jnp.exp(o) - 1.0)
    mm = jnp.max(o, axis=1, keepdims=True)
    z = o - mm
    lse = jnp.log(jnp.sum(jnp.exp(z), axis=1, keepdims=True))
    out_ref[...] = z - lse


def kernel(x, adj, W_heads, a_src_heads, a_dst_heads, b_heads,
           W_out, a_src_out, a_dst_out, b_out):
    adj = adj.astype(jnp.int32)
    njb = N // BJ

    Wcat = jnp.transpose(W_heads, (1, 0, 2)).reshape(IN_FEAT, FCAT)
    eye = jnp.eye(N_HEADS, dtype=jnp.float32)
    Asrc = (a_src_heads[:, :, None] * eye[:, None, :]).reshape(FCAT, N_HEADS)
    Adst = (a_dst_heads[:, :, None] * eye[:, None, :]).reshape(FCAT, N_HEADS)

    hproj, s_all, d_allT = pl.pallas_call(
        _prep_kernel,
        in_specs=[
            pl.BlockSpec((N, IN_FEAT), lambda: (0, 0)),
            pl.BlockSpec((IN_FEAT, FCAT), lambda: (0, 0)),
            pl.BlockSpec((FCAT, N_HEADS), lambda: (0, 0)),
            pl.BlockSpec((FCAT, N_HEADS), lambda: (0, 0)),
        ],
        out_specs=[
            pl.BlockSpec((N, FCAT), lambda: (0, 0)),
            pl.BlockSpec((N, N_HEADS), lambda: (0, 0)),
            pl.BlockSpec((N_HEADS, N), lambda: (0, 0)),
        ],
        out_shape=[
            jax.ShapeDtypeStruct((N, FCAT), jnp.float32),
            jax.ShapeDtypeStruct((N, N_HEADS), jnp.float32),
            jax.ShapeDtypeStruct((N_HEADS, N), jnp.float32),
        ],
    )(x, Wcat, Asrc, Adst)

    hcatT = pl.pallas_call(
        _layer1_kernel,
        grid=(njb,),
        in_specs=[
            pl.BlockSpec((N, FCAT), lambda jb: (0, 0)),
            pl.BlockSpec((N, N_HEADS), lambda jb: (0, 0)),
            pl.BlockSpec((N_HEADS, BJ), lambda jb: (0, jb)),
            pl.BlockSpec((N, BJ), lambda jb: (0, jb)),
            pl.BlockSpec((N_HIDDEN, N_HEADS), lambda jb: (0, 0)),
        ],
        out_specs=pl.BlockSpec((FCAT, BJ), lambda jb: (0, jb)),
        out_shape=jax.ShapeDtypeStruct((FCAT, N), jnp.float32),
    )(hproj, s_all, d_allT, adj, jnp.transpose(b_heads))

    out = pl.pallas_call(
        _layer2_kernel,
        grid=(njb,),
        in_specs=[
            pl.BlockSpec((FCAT, N), lambda jb: (0, 0)),
            pl.BlockSpec((FCAT, BJ), lambda jb: (0, jb)),
            pl.BlockSpec((N, BJ), lambda jb: (0, jb)),
            pl.BlockSpec((FCAT, OUT_FEAT), lambda jb: (0, 0)),
            pl.BlockSpec((1, OUT_FEAT), lambda jb: (0, 0)),
            pl.BlockSpec((1, OUT_FEAT), lambda jb: (0, 0)),
            pl.BlockSpec((1, OUT_FEAT), lambda jb: (0, 0)),
        ],
        out_specs=pl.BlockSpec((BJ, OUT_FEAT), lambda jb: (jb, 0)),
        out_shape=jax.ShapeDtypeStruct((N, OUT_FEAT), jnp.float32),
    )(hcatT, hcatT, adj, W_out,
      a_src_out.reshape(1, OUT_FEAT), a_dst_out.reshape(1, OUT_FEAT),
      b_out.reshape(1, OUT_FEAT))
    return out


# speedup vs baseline: 16668.1805x; 1.2795x over previous
import jax
import jax.numpy as jnp
from jax.experimental import pallas as pl
from jax.experimental.pallas import tpu as pltpu

N = 1024
IN_FEAT = 128
N_HIDDEN = 64
N_HEADS = 8
FCAT = N_HIDDEN * N_HEADS
OUT_FEAT = 64
NEG_SLOPE = 0.2


def _mask01(adj):
    ii = jax.lax.broadcasted_iota(jnp.int32, (N, N), 0)
    jj = jax.lax.broadcasted_iota(jnp.int32, (N, N), 1)
    valid = ((adj != 0) & (ii != jj)) | (ii == jj)
    return jnp.where(valid, 1.0, 0.0)


def _prob(s, d, mask01):
    smax = jnp.max(s, axis=0, keepdims=True)
    t = smax + d
    mrow = jnp.where(t > 0, t, NEG_SLOPE * t)
    d1 = d - mrow
    d2 = NEG_SLOPE * d - mrow
    p = jnp.exp(jnp.maximum(s + d1, NEG_SLOPE * s + d2))
    return p * mask01


def _fused_kernel(x_ref, adj_ref, Wcat_ref, Asrc_ref, Adst_ref, bT_ref,
                  Wout_ref, as2_ref, ad2_ref, b2_ref, out_ref, hcatT_scr):
    hproj = jnp.dot(x_ref[...], Wcat_ref[...],
                    preferred_element_type=jnp.float32)
    s_all = jnp.dot(hproj, Asrc_ref[...],
                    preferred_element_type=jnp.float32)
    d_allT = jax.lax.dot_general(
        Adst_ref[...], hproj, (((0,), (1,)), ((), ())),
        preferred_element_type=jnp.float32)
    mask01 = _mask01(adj_ref[...])

    for k in range(N_HEADS):
        s = s_all[:, k:k + 1]
        d = d_allT[k:k + 1, :]
        p = _prob(s, d, mask01)
        denom = jnp.sum(p, axis=0, keepdims=True)
        h = hproj[:, k * N_HIDDEN:(k + 1) * N_HIDDEN]
        accT = jax.lax.dot_general(h, p, (((0,), (0,)), ((), ())),
                                   preferred_element_type=jnp.float32)
        outT = accT * (1.0 / (denom + 1e-16)) + bT_ref[:, k:k + 1]
        hcatT_scr[k * N_HIDDEN:(k + 1) * N_HIDDEN, :] = outT

    hcT = hcatT_scr[...]
    h2 = jax.lax.dot_general(hcT, Wout_ref[...], (((0,), (0,)), ((), ())),
                             preferred_element_type=jnp.float32)
    s2 = jax.lax.dot_general(h2, as2_ref[...], (((1,), (1,)), ((), ())),
                             preferred_element_type=jnp.float32)
    d2 = jax.lax.dot_general(ad2_ref[...], h2, (((1,), (1,)), ((), ())),
                             preferred_element_type=jnp.float32)
    p = _prob(s2, d2, mask01)
    denom = jnp.sum(p, axis=0, keepdims=True)
    acc = jax.lax.dot_general(p, h2, (((0,), (0,)), ((), ())),
                              preferred_element_type=jnp.float32)
    recip_col = jnp.transpose(1.0 / (denom + 1e-16))
    o = acc * recip_col + b2_ref[...]
    o = jnp.where(o > 0, o, jnp.exp(o) - 1.0)
    mm = jnp.max(o, axis=1, keepdims=True)
    z = o - mm
    lse = jnp.log(jnp.sum(jnp.exp(z), axis=1, keepdims=True))
    out_ref[...] = z - lse


def _full_spec(*shape):
    n = len(shape)
    return pl.BlockSpec(shape, lambda: (0,) * n)


def kernel(x, adj, W_heads, a_src_heads, a_dst_heads, b_heads,
           W_out, a_src_out, a_dst_out, b_out):
    adj = adj.astype(jnp.int32)

    Wcat = jnp.transpose(W_heads, (1, 0, 2)).reshape(IN_FEAT, FCAT)
    eye = jnp.eye(N_HEADS, dtype=jnp.float32)
    Asrc = (a_src_heads[:, :, None] * eye[:, None, :]).reshape(FCAT, N_HEADS)
    Adst = (a_dst_heads[:, :, None] * eye[:, None, :]).reshape(FCAT, N_HEADS)

    out = pl.pallas_call(
        _fused_kernel,
        in_specs=[
            _full_spec(N, IN_FEAT),
            _full_spec(N, N),
            _full_spec(IN_FEAT, FCAT),
            _full_spec(FCAT, N_HEADS),
            _full_spec(FCAT, N_HEADS),
            _full_spec(N_HIDDEN, N_HEADS),
            _full_spec(FCAT, OUT_FEAT),
            _full_spec(1, OUT_FEAT),
            _full_spec(1, OUT_FEAT),
            _full_spec(1, OUT_FEAT),
        ],
        out_specs=_full_spec(N, OUT_FEAT),
        out_shape=jax.ShapeDtypeStruct((N, OUT_FEAT), jnp.float32),
        scratch_shapes=[pltpu.VMEM((FCAT, N), jnp.float32)],
    )(x, adj, Wcat, Asrc, Adst, jnp.transpose(b_heads), W_out,
      a_src_out.reshape(1, OUT_FEAT), a_dst_out.reshape(1, OUT_FEAT),
      b_out.reshape(1, OUT_FEAT))
    return out


# speedup vs baseline: 19727.3272x; 1.1835x over previous
import jax
import jax.numpy as jnp
from jax.experimental import pallas as pl
from jax.experimental.pallas import tpu as pltpu

N = 1024
IN_FEAT = 128
N_HIDDEN = 64
N_HEADS = 8
FCAT = N_HIDDEN * N_HEADS
OUT_FEAT = 64
NEG_SLOPE = 0.2


def _mask01(adj):
    ii = jax.lax.broadcasted_iota(jnp.int32, (N, N), 0)
    jj = jax.lax.broadcasted_iota(jnp.int32, (N, N), 1)
    valid = ((adj != 0) & (ii != jj)) | (ii == jj)
    return jnp.where(valid, 1.0, 0.0)


def _row_factors(smax, d):
    t = smax + d
    mrow = jnp.where(t > 0, t, NEG_SLOPE * t)
    B = jnp.exp(d + smax - mrow)
    D = jnp.exp(NEG_SLOPE * (d + smax) - mrow)
    return B, D


def _attend(A, B, C, D, mask01, h_ext):
    p = jnp.maximum(A * B, C * D) * mask01
    return jax.lax.dot_general(h_ext, p, (((0,), (0,)), ((), ())),
                               preferred_element_type=jnp.float32)


def _fused_kernel(x_ref, adj_ref, Wcat_ref, Asrc_ref, Adst_ref, bT_ref,
                  Wout_ref, as2_ref, ad2_ref, b2_ref, out_ref, hcatT_scr):
    hproj = jnp.dot(x_ref[...], Wcat_ref[...],
                    preferred_element_type=jnp.float32)
    s_all = jnp.dot(hproj, Asrc_ref[...],
                    preferred_element_type=jnp.float32)
    d_allT = jax.lax.dot_general(
        Adst_ref[...], hproj, (((0,), (1,)), ((), ())),
        preferred_element_type=jnp.float32)
    mask01 = _mask01(adj_ref[...])
    ones_col = jnp.ones((N, 1), dtype=jnp.float32)

    smax_all = jnp.max(s_all, axis=0, keepdims=True)
    A_all = jnp.exp(s_all - smax_all)
    C_all = jnp.exp(NEG_SLOPE * (s_all - smax_all))

    for k in range(N_HEADS):
        B, D = _row_factors(smax_all[:, k:k + 1], d_allT[k:k + 1, :])
        h_ext = jnp.concatenate(
            [hproj[:, k * N_HIDDEN:(k + 1) * N_HIDDEN], ones_col], axis=1)
        accT = _attend(A_all[:, k:k + 1], B, C_all[:, k:k + 1], D,
                       mask01, h_ext)
        recip = 1.0 / (accT[N_HIDDEN:N_HIDDEN + 1, :] + 1e-16)
        outT = accT[:N_HIDDEN, :] * recip + bT_ref[:, k:k + 1]
        hcatT_scr[k * N_HIDDEN:(k + 1) * N_HIDDEN, :] = outT

    hcT = hcatT_scr[...]
    h2 = jax.lax.dot_general(hcT, Wout_ref[...], (((0,), (0,)), ((), ())),
                             preferred_element_type=jnp.float32)
    s2 = jax.lax.dot_general(h2, as2_ref[...], (((1,), (1,)), ((), ())),
                             preferred_element_type=jnp.float32)
    d2 = jax.lax.dot_general(ad2_ref[...], h2, (((1,), (1,)), ((), ())),
                             preferred_element_type=jnp.float32)
    smax2 = jnp.max(s2, axis=0, keepdims=True)
    A2 = jnp.exp(s2 - smax2)
    C2 = jnp.exp(NEG_SLOPE * (s2 - smax2))
    B2, D2 = _row_factors(smax2, d2)
    h2_ext = jnp.concatenate([h2, ones_col], axis=1)
    accT2 = _attend(A2, B2, C2, D2, mask01, h2_ext)
    recip2 = 1.0 / (accT2[OUT_FEAT:OUT_FEAT + 1, :] + 1e-16)
    oT = accT2[:OUT_FEAT, :] * recip2 + jnp.transpose(b2_ref[...])
    oT = jnp.where(oT > 0, oT, jnp.exp(oT) - 1.0)
    mm = jnp.max(oT, axis=0, keepdims=True)
    z = oT - mm
    lse = jnp.log(jnp.sum(jnp.exp(z), axis=0, keepdims=True))
    out_ref[...] = jnp.transpose(z - lse)


def _full_spec(*shape):
    n = len(shape)
    return pl.BlockSpec(shape, lambda: (0,) * n)


def kernel(x, adj, W_heads, a_src_heads, a_dst_heads, b_heads,
           W_out, a_src_out, a_dst_out, b_out):
    adj = adj.astype(jnp.int32)

    Wcat = jnp.transpose(W_heads, (1, 0, 2)).reshape(IN_FEAT, FCAT)
    eye = jnp.eye(N_HEADS, dtype=jnp.float32)
    Asrc = (a_src_heads[:, :, None] * eye[:, None, :]).reshape(FCAT, N_HEADS)
    Adst = (a_dst_heads[:, :, None] * eye[:, None, :]).reshape(FCAT, N_HEADS)

    out = pl.pallas_call(
        _fused_kernel,
        in_specs=[
            _full_spec(N, IN_FEAT),
            _full_spec(N, N),
            _full_spec(IN_FEAT, FCAT),
            _full_spec(FCAT, N_HEADS),
            _full_spec(FCAT, N_HEADS),
            _full_spec(N_HIDDEN, N_HEADS),
            _full_spec(FCAT, OUT_FEAT),
            _full_spec(1, OUT_FEAT),
            _full_spec(1, OUT_FEAT),
            _full_spec(1, OUT_FEAT),
        ],
        out_specs=_full_spec(N, OUT_FEAT),
        out_shape=jax.ShapeDtypeStruct((N, OUT_FEAT), jnp.float32),
        scratch_shapes=[pltpu.VMEM((FCAT, N), jnp.float32)],
    )(x, adj, Wcat, Asrc, Adst, jnp.transpose(b_heads), W_out,
      a_src_out.reshape(1, OUT_FEAT), a_dst_out.reshape(1, OUT_FEAT),
      b_out.reshape(1, OUT_FEAT))
    return out


# speedup vs baseline: 20805.1081x; 1.0546x over previous
import jax
import jax.numpy as jnp
from jax.experimental import pallas as pl
from jax.experimental.pallas import tpu as pltpu

N = 1024
IN_FEAT = 128
N_HIDDEN = 64
N_HEADS = 8
FCAT = N_HIDDEN * N_HEADS
OUT_FEAT = 64
NEG_SLOPE = 0.2


def _mask01(adj):
    ii = jax.lax.broadcasted_iota(jnp.int32, (N, N), 0)
    jj = jax.lax.broadcasted_iota(jnp.int32, (N, N), 1)
    valid = ((adj != 0) & (ii != jj)) | (ii == jj)
    return jnp.where(valid, 1.0, 0.0)


def _attend(A, C, E, mask01, h_ext):
    q = jnp.maximum(A, C * E) * mask01
    return jax.lax.dot_general(h_ext, q, (((0,), (0,)), ((), ())),
                               preferred_element_type=jnp.float32)


def _fused_kernel(x_ref, adj_ref, Wcat_ref, Asrc_ref, Adst_ref, bT_ref,
                  Wout_ref, as2_ref, ad2_ref, b2_ref, out_ref, hcatT_scr):
    hproj = jnp.dot(x_ref[...], Wcat_ref[...],
                    preferred_element_type=jnp.float32)
    s_all = jnp.dot(hproj, Asrc_ref[...],
                    preferred_element_type=jnp.float32)
    d_allT = jax.lax.dot_general(
        Adst_ref[...], hproj, (((0,), (1,)), ((), ())),
        preferred_element_type=jnp.float32)
    mask01 = _mask01(adj_ref[...])
    ones_col = jnp.ones((N, 1), dtype=jnp.float32)

    smax_all = jnp.max(s_all, axis=0, keepdims=True)
    A_all = jnp.exp(s_all - smax_all)
    C_all = jnp.exp(NEG_SLOPE * (s_all - smax_all))

    for k in range(N_HEADS):
        E = jnp.exp(-(1.0 - NEG_SLOPE) *
                    (d_allT[k:k + 1, :] + smax_all[:, k:k + 1]))
        h_ext = jnp.concatenate(
            [hproj[:, k * N_HIDDEN:(k + 1) * N_HIDDEN], ones_col], axis=1)
        accT = _attend(A_all[:, k:k + 1], C_all[:, k:k + 1], E,
                       mask01, h_ext)
        recip = 1.0 / (accT[N_HIDDEN:N_HIDDEN + 1, :] + 1e-16)
        outT = accT[:N_HIDDEN, :] * recip + bT_ref[:, k:k + 1]
        hcatT_scr[k * N_HIDDEN:(k + 1) * N_HIDDEN, :] = outT

    hcT = hcatT_scr[...]
    h2 = jax.lax.dot_general(hcT, Wout_ref[...], (((0,), (0,)), ((), ())),
                             preferred_element_type=jnp.float32)
    s2 = jax.lax.dot_general(h2, as2_ref[...], (((1,), (1,)), ((), ())),
                             preferred_element_type=jnp.float32)
    d2 = jax.lax.dot_general(ad2_ref[...], h2, (((1,), (1,)), ((), ())),
                             preferred_element_type=jnp.float32)
    smax2 = jnp.max(s2, axis=0, keepdims=True)
    A2 = jnp.exp(s2 - smax2)
    C2 = jnp.exp(NEG_SLOPE * (s2 - smax2))
    E2 = jnp.exp(-(1.0 - NEG_SLOPE) * (d2 + smax2))
    h2_ext = jnp.concatenate([h2, ones_col], axis=1)
    accT2 = _attend(A2, C2, E2, mask01, h2_ext)
    recip2 = 1.0 / (accT2[OUT_FEAT:OUT_FEAT + 1, :] + 1e-16)
    oT = accT2[:OUT_FEAT, :] * recip2 + jnp.transpose(b2_ref[...])
    oT = jnp.where(oT > 0, oT, jnp.exp(oT) - 1.0)
    mm = jnp.max(oT, axis=0, keepdims=True)
    z = oT - mm
    lse = jnp.log(jnp.sum(jnp.exp(z), axis=0, keepdims=True))
    out_ref[...] = jnp.transpose(z - lse)


def _full_spec(*shape):
    n = len(shape)
    return pl.BlockSpec(shape, lambda: (0,) * n)


def kernel(x, adj, W_heads, a_src_heads, a_dst_heads, b_heads,
           W_out, a_src_out, a_dst_out, b_out):
    adj = adj.astype(jnp.int32)

    Wcat = jnp.transpose(W_heads, (1, 0, 2)).reshape(IN_FEAT, FCAT)
    eye = jnp.eye(N_HEADS, dtype=jnp.float32)
    Asrc = (a_src_heads[:, :, None] * eye[:, None, :]).reshape(FCAT, N_HEADS)
    Adst = (a_dst_heads[:, :, None] * eye[:, None, :]).reshape(FCAT, N_HEADS)

    out = pl.pallas_call(
        _fused_kernel,
        in_specs=[
            _full_spec(N, IN_FEAT),
            _full_spec(N, N),
            _full_spec(IN_FEAT, FCAT),
            _full_spec(FCAT, N_HEADS),
            _full_spec(FCAT, N_HEADS),
            _full_spec(N_HIDDEN, N_HEADS),
            _full_spec(FCAT, OUT_FEAT),
            _full_spec(1, OUT_FEAT),
            _full_spec(1, OUT_FEAT),
            _full_spec(1, OUT_FEAT),
        ],
        out_specs=_full_spec(N, OUT_FEAT),
        out_shape=jax.ShapeDtypeStruct((N, OUT_FEAT), jnp.float32),
        scratch_shapes=[pltpu.VMEM((FCAT, N), jnp.float32)],
    )(x, adj, Wcat, Asrc, Adst, jnp.transpose(b_heads), W_out,
      a_src_out.reshape(1, OUT_FEAT), a_dst_out.reshape(1, OUT_FEAT),
      b_out.reshape(1, OUT_FEAT))
    return out
